# Initial kernel scaffold; baseline (speedup 1.0000x reference)
import jax
import jax.numpy as jnp
from jax.experimental import pallas as pl


def kernel(out, W_write, W_read, W_beta, decay):
    raise NotImplementedError("write your pallas kernel here")


# speedup vs baseline: 1.0667x; 1.0667x over previous
import functools

import jax
import jax.numpy as jnp
from jax.experimental import pallas as pl
from jax.experimental.pallas import tpu as pltpu

C = 64
G = 1
TM = 512

_dot = functools.partial(jax.lax.dot_general,
                         preferred_element_type=jnp.float32,
                         precision=jax.lax.Precision.HIGHEST)


def _mm(a, b):
    return _dot(a, b, (((1,), (0,)), ((), ())))


def _mmT(a, b):
    return _dot(a, b, (((1,), (1,)), ((), ())))


def _mTm(a, b):
    return _dot(a, b, (((0,), (0,)), ((), ())))


def _proj_in_body(x_ref, ww_ref, wb_ref, v_ref, beta_ref):
    x = x_ref[...]
    v_ref[...] = _mmT(x, ww_ref[...])
    beta_ref[...] = jax.nn.sigmoid(_mmT(x, wb_ref[...]))


def _proj_out_body(o_ref, out_ref, wr_ref, y_ref):
    y_ref[...] = out_ref[...] + _mmT(o_ref[...], wr_ref[...])


def _scan_body(x_ref, v_ref, beta_ref, L_ref, dexp_ref, dw_ref, ct_ref,
               o_ref, S_ref, prev_ref, *, H, d):
    n = pl.program_id(1)

    @pl.when(n == 0)
    def _():
        S_ref[...] = jnp.zeros_like(S_ref)
        prev_ref[...] = jnp.zeros_like(prev_ref)

    ii = jax.lax.broadcasted_iota(jnp.int32, (C, C), 0)
    jj = jax.lax.broadcasted_iota(jnp.int32, (C, C), 1)
    strict = ii > jj
    eye = jnp.where(ii == jj, 1.0, 0.0).astype(jnp.float32)

    hg = jax.lax.rem(pl.program_id(0), jnp.int32(H // G))

    for g in range(G):
        sl = slice(g * d, (g + 1) * d)
        xh = x_ref[:, sl]
        n2 = jnp.sum(xh * xh, axis=1, keepdims=True)
        rk = xh / jnp.maximum(jnp.sqrt(n2), 1e-12)
        prev = prev_ref[g:g + 1, :]
        wk = jnp.concatenate([prev, rk[:C - 1, :]], axis=0)
        prev_ref[g:g + 1, :] = rk[C - 1:C, :]

        beta = beta_ref[g].reshape(C, 1)
        L = L_ref[g]
        dexp = dexp_ref[g].reshape(C, 1)
        dwv = dw_ref[g].reshape(C, 1)
        ct = ct_ref[hg * G + g]

        vb = v_ref[:, sl] * beta
        wkb = wk * beta

        M = jnp.where(strict, -_mmT(wkb, wk) * L, 0.0)
        P = M + eye
        Mk = M
        for _ in range(5):
            Mk = _mm(Mk, Mk)
            P = P + _mm(P, Mk)

        v_c = _mm(P, vb)
        wk_cum = _mm(P, wkb * dexp)
        attn = _mmT(rk, wk) * L

        S = S_ref[sl, :]
        v_new = v_c - _mm(wk_cum, S)
        o_ref[:, sl] = _mm(rk * dexp, S) + _mm(attn, v_new)
        S_ref[sl, :] = ct * S + _mTm(wk * dwv, v_new)


def kernel(out, W_write, W_read, W_beta, decay):
    B, T, D = out.shape
    H = decay.shape[0]
    d = D // H
    N = T // C
    BT = B * T
    f32 = jnp.float32

    x2 = out.reshape(BT, D).astype(f32)

    log_gamma = jnp.log(jax.nn.sigmoid(decay))
    pos = jnp.arange(C, dtype=f32)
    cum = (pos + 1.0) * log_gamma[:, None]
    tril = jnp.tril(jnp.ones((C, C), f32))
    L_mask = jnp.exp((cum[:, :, None] - cum[:, None, :]) * tril) * tril
    decay_exp = jnp.exp(cum).reshape(H, C, 1)
    chunk_total = jnp.exp(cum[:, -1])
    dw = jnp.exp(cum[:, -1:] - cum).reshape(H, C, 1)

    n_tiles = BT // TM
    v_flat, beta_flat = pl.pallas_call(
        _proj_in_body,
        grid=(n_tiles,),
        in_specs=[
            pl.BlockSpec((TM, D), lambda i: (i, 0)),
            pl.BlockSpec((D, D), lambda i: (0, 0)),
            pl.BlockSpec((H, D), lambda i: (0, 0)),
        ],
        out_specs=[
            pl.BlockSpec((TM, D), lambda i: (i, 0)),
            pl.BlockSpec((TM, H), lambda i: (i, 0)),
        ],
        out_shape=[
            jax.ShapeDtypeStruct((BT, D), f32),
            jax.ShapeDtypeStruct((BT, H), f32),
        ],
        compiler_params=pltpu.CompilerParams(
            dimension_semantics=("parallel",)),
        name="hebbian_proj_in",
    )(x2, W_write, W_beta)

    betaT = beta_flat.reshape(B, N, C, H).transpose(0, 3, 1, 2) \
                     .reshape(B * H, N, C, 1)

    BH = B * H
    scan_body = functools.partial(_scan_body, H=H, d=d)
    o_flat = pl.pallas_call(
        scan_body,
        grid=(BH // G, N),
        in_specs=[
            pl.BlockSpec((C, G * d), lambda p, n: (p // (H // G) * N + n,
                                                   jax.lax.rem(p, H // G))),
            pl.BlockSpec((C, G * d), lambda p, n: (p // (H // G) * N + n,
                                                   jax.lax.rem(p, H // G))),
            pl.BlockSpec((G, 1, C, 1), lambda p, n: (p, n, 0, 0)),
            pl.BlockSpec((G, C, C), lambda p, n: (jax.lax.rem(p, H // G), 0, 0)),
            pl.BlockSpec((G, C, 1), lambda p, n: (jax.lax.rem(p, H // G), 0, 0)),
            pl.BlockSpec((G, C, 1), lambda p, n: (jax.lax.rem(p, H // G), 0, 0)),
            pl.BlockSpec(memory_space=pltpu.SMEM),
        ],
        out_specs=pl.BlockSpec((C, G * d), lambda p, n: (p // (H // G) * N + n,
                                                         jax.lax.rem(p, H // G))),
        out_shape=jax.ShapeDtypeStruct((BT, D), f32),
        scratch_shapes=[
            pltpu.VMEM((G * d, d), f32),
            pltpu.VMEM((8, d), f32),
        ],
        compiler_params=pltpu.CompilerParams(
            dimension_semantics=("parallel", "arbitrary")),
        name="hebbian_scan",
    )(x2, v_flat, betaT, L_mask, decay_exp, dw, chunk_total)

    y = pl.pallas_call(
        _proj_out_body,
        grid=(n_tiles,),
        in_specs=[
            pl.BlockSpec((TM, D), lambda i: (i, 0)),
            pl.BlockSpec((TM, D), lambda i: (i, 0)),
            pl.BlockSpec((D, D), lambda i: (0, 0)),
        ],
        out_specs=pl.BlockSpec((TM, D), lambda i: (i, 0)),
        out_shape=jax.ShapeDtypeStruct((BT, D), f32),
        compiler_params=pltpu.CompilerParams(
            dimension_semantics=("parallel",)),
        name="hebbian_proj_out",
    )(o_flat, x2, W_read)

    return y.reshape(B, T, D).astype(out.dtype)


# speedup vs baseline: 7.9764x; 7.4774x over previous
import functools

import jax
import jax.numpy as jnp
from jax.experimental import pallas as pl
from jax.experimental.pallas import tpu as pltpu

C = 64
G = 8
TM = 512

_dot = functools.partial(jax.lax.dot_general,
                         preferred_element_type=jnp.float32,
                         precision=None)


def _mm(a, b):
    return _dot(a, b, (((1,), (0,)), ((), ())))


def _mmT(a, b):
    return _dot(a, b, (((1,), (1,)), ((), ())))


def _mTm(a, b):
    return _dot(a, b, (((0,), (0,)), ((), ())))


def _proj_in_body(x_ref, ww_ref, wb_ref, e_ref, v_ref, bexp_ref):
    x = x_ref[...]
    bexp = _mm(jax.nn.sigmoid(_mmT(x, wb_ref[...])), e_ref[...])
    bexp_ref[...] = bexp
    v_ref[...] = _mmT(x, ww_ref[...]) * bexp


def _proj_out_body(o_ref, out_ref, wr_ref, y_ref):
    y_ref[...] = out_ref[...] + _mmT(o_ref[...], wr_ref[...])


def _scan_body(x_ref, v_ref, bexp_ref, L_ref, dexp_ref, dw_ref, ct_ref,
               o_ref, S_ref, prev_ref, *, H, d):
    n = pl.program_id(1)

    @pl.when(n == 0)
    def _():
        S_ref[...] = jnp.zeros_like(S_ref)
        prev_ref[...] = jnp.zeros_like(prev_ref)

    ii = jax.lax.broadcasted_iota(jnp.int32, (C, C), 0)
    jj = jax.lax.broadcasted_iota(jnp.int32, (C, C), 1)
    strict = ii > jj

    hg = jax.lax.rem(pl.program_id(0), jnp.int32(H // G))

    S_all = S_ref[...]
    prev_all = prev_ref[...]
    x_all = x_ref[...]
    v_all = v_ref[...]
    b_all = bexp_ref[...]
    L_all = L_ref[...]
    de_all = dexp_ref[...]
    dw_all = dw_ref[...]

    rk_l, wk_l, M_l, X_l, attn_l = [], [], [], [], []
    new_S, new_prev = [], []
    for g in range(G):
        sl = slice(g * d, (g + 1) * d)
        xh = x_all[:, sl]
        n2 = jnp.sum(xh * xh, axis=1, keepdims=True)
        rk = xh * jax.lax.rsqrt(jnp.maximum(n2, 1e-24))
        wk = jnp.concatenate([prev_all[g:g + 1, :], rk[:C - 1, :]], axis=0)
        new_prev.append(rk[C - 1:C, :])
        rk_l.append(rk)
        wk_l.append(wk)
    for g in range(G):
        sl = slice(g * d, (g + 1) * d)
        wkb = wk_l[g] * b_all[:, sl]
        M_l.append(jnp.where(strict, -_mmT(wkb, wk_l[g]) * L_all[g], 0.0))
        X_l.append(jnp.concatenate([v_all[:, sl], wkb * de_all[g]], axis=1))
    for g in range(G):
        attn_l.append(_mmT(rk_l[g], wk_l[g]) * L_all[g])
    Mk_l = list(M_l)
    for g in range(G):
        X_l[g] = X_l[g] + _mm(Mk_l[g], X_l[g])
    for _ in range(5):
        for g in range(G):
            Mk_l[g] = _mm(Mk_l[g], Mk_l[g])
        for g in range(G):
            X_l[g] = X_l[g] + _mm(Mk_l[g], X_l[g])
    vn_l = []
    for g in range(G):
        sl = slice(g * d, (g + 1) * d)
        vn_l.append(X_l[g][:, :d] - _mm(X_l[g][:, d:], S_all[sl, :]))
    for g in range(G):
        sl = slice(g * d, (g + 1) * d)
        o_ref[:, sl] = (_mm(rk_l[g] * de_all[g], S_all[sl, :])
                        + _mm(attn_l[g], vn_l[g]))
    for g in range(G):
        sl = slice(g * d, (g + 1) * d)
        new_S.append(ct_ref[hg * G + g] * S_all[sl, :]
                     + _mTm(wk_l[g] * dw_all[g], vn_l[g]))

    S_ref[...] = jnp.concatenate(new_S, axis=0)
    prev_ref[0:G, :] = jnp.concatenate(new_prev, axis=0)


def kernel(out, W_write, W_read, W_beta, decay):
    B, T, D = out.shape
    H = decay.shape[0]
    d = D // H
    N = T // C
    BT = B * T
    f32 = jnp.float32

    x2 = out.reshape(BT, D).astype(f32)

    log_gamma = jnp.log(jax.nn.sigmoid(decay))
    pos = jnp.arange(C, dtype=f32)
    cum = (pos + 1.0) * log_gamma[:, None]
    tril = jnp.tril(jnp.ones((C, C), f32))
    L_mask = jnp.exp((cum[:, :, None] - cum[:, None, :]) * tril) * tril
    decay_exp = jnp.broadcast_to(jnp.exp(cum)[:, :, None], (H, C, d)) + 0.0
    chunk_total = jnp.exp(cum[:, -1])
    dw = jnp.broadcast_to(jnp.exp(cum[:, -1:] - cum)[:, :, None], (H, C, d)) + 0.0
    eh = jnp.repeat(jnp.eye(H, dtype=f32), d, axis=1)

    n_tiles = BT // TM
    v_flat, bexp_flat = pl.pallas_call(
        _proj_in_body,
        grid=(n_tiles,),
        in_specs=[
            pl.BlockSpec((TM, D), lambda i: (i, 0)),
            pl.BlockSpec((D, D), lambda i: (0, 0)),
            pl.BlockSpec((H, D), lambda i: (0, 0)),
            pl.BlockSpec((H, D), lambda i: (0, 0)),
        ],
        out_specs=[
            pl.BlockSpec((TM, D), lambda i: (i, 0)),
            pl.BlockSpec((TM, D), lambda i: (i, 0)),
        ],
        out_shape=[
            jax.ShapeDtypeStruct((BT, D), f32),
            jax.ShapeDtypeStruct((BT, D), f32),
        ],
        compiler_params=pltpu.CompilerParams(
            dimension_semantics=("parallel",)),
        name="hebbian_proj_in",
    )(x2, W_write, W_beta, eh)

    BH = B * H
    scan_body = functools.partial(_scan_body, H=H, d=d)
    row_map = lambda p, n: (p // (H // G) * N + n, jax.lax.rem(p, H // G))
    hd_map = lambda p, n: (jax.lax.rem(p, H // G), 0, 0)
    o_flat = pl.pallas_call(
        scan_body,
        grid=(BH // G, N),
        in_specs=[
            pl.BlockSpec((C, G * d), row_map),
            pl.BlockSpec((C, G * d), row_map),
            pl.BlockSpec((C, G * d), row_map),
            pl.BlockSpec((G, C, C), hd_map),
            pl.BlockSpec((G, C, d), hd_map),
            pl.BlockSpec((G, C, d), hd_map),
            pl.BlockSpec(memory_space=pltpu.SMEM),
        ],
        out_specs=pl.BlockSpec((C, G * d), row_map),
        out_shape=jax.ShapeDtypeStruct((BT, D), f32),
        scratch_shapes=[
            pltpu.VMEM((G * d, d), f32),
            pltpu.VMEM((8, d), f32),
        ],
        compiler_params=pltpu.CompilerParams(
            dimension_semantics=("parallel", "arbitrary")),
        name="hebbian_scan",
    )(x2, v_flat, bexp_flat, L_mask, decay_exp, dw, chunk_total)

    y = pl.pallas_call(
        _proj_out_body,
        grid=(n_tiles,),
        in_specs=[
            pl.BlockSpec((TM, D), lambda i: (i, 0)),
            pl.BlockSpec((TM, D), lambda i: (i, 0)),
            pl.BlockSpec((D, D), lambda i: (0, 0)),
        ],
        out_specs=pl.BlockSpec((TM, D), lambda i: (i, 0)),
        out_shape=jax.ShapeDtypeStruct((BT, D), f32),
        compiler_params=pltpu.CompilerParams(
            dimension_semantics=("parallel",)),
        name="hebbian_proj_out",
    )(o_flat, x2, W_read)

    return y.reshape(B, T, D).astype(out.dtype)


# speedup vs baseline: 10.9166x; 1.3686x over previous
import functools

import jax
import jax.numpy as jnp
from jax.experimental import pallas as pl
from jax.experimental.pallas import tpu as pltpu

C = 64
G = 8
CPB = 4
TM = 512

_dot = functools.partial(jax.lax.dot_general,
                         preferred_element_type=jnp.float32,
                         precision=None)


def _mm(a, b):
    return _dot(a, b, (((1,), (0,)), ((), ())))


def _mmT(a, b):
    return _dot(a, b, (((1,), (1,)), ((), ())))


def _mTm(a, b):
    return _dot(a, b, (((0,), (0,)), ((), ())))


def _proj_in_body(x_ref, ww_ref, wb_ref, e_ref, v_ref, bexp_ref):
    x = x_ref[...]
    bexp = _mm(jax.nn.sigmoid(_mmT(x, wb_ref[...])), e_ref[...])
    bexp_ref[...] = bexp
    v_ref[...] = _mmT(x, ww_ref[...]) * bexp


def _proj_out_body(o_ref, out_ref, wr_ref, y_ref):
    y_ref[...] = out_ref[...] + _mmT(o_ref[...], wr_ref[...])


def _scan_body(x_ref, v_ref, bexp_ref, L_ref, dexp_ref, dw_ref, ct_ref,
               o_ref, S_ref, prev_ref, *, H, d):
    n = pl.program_id(1)

    @pl.when(n == 0)
    def _():
        S_ref[...] = jnp.zeros_like(S_ref)
        prev_ref[...] = jnp.zeros_like(prev_ref)

    ii = jax.lax.broadcasted_iota(jnp.int32, (C, C), 0)
    jj = jax.lax.broadcasted_iota(jnp.int32, (C, C), 1)
    strict = ii > jj

    hg = jax.lax.rem(pl.program_id(0), jnp.int32(H // G))

    S_all = S_ref[...]
    prev_all = prev_ref[...]
    x_all = x_ref[...]
    v_all = v_ref[...]
    b_all = bexp_ref[...]
    L_all = L_ref[...]
    de_all = dexp_ref[...]
    dw_all = dw_ref[...]

    rk_l, wk_l, Mk_l, X_l, attn_l = {}, {}, {}, {}, {}
    for c in range(CPB):
        rs = slice(c * C, (c + 1) * C)
        for g in range(G):
            sl = slice(g * d, (g + 1) * d)
            xh = x_all[rs, sl]
            n2 = jnp.sum(xh * xh, axis=1, keepdims=True)
            rk_l[c, g] = xh * jax.lax.rsqrt(jnp.maximum(n2, 1e-24))
    for c in range(CPB):
        for g in range(G):
            prev = (prev_all[g:g + 1, :] if c == 0
                    else rk_l[c - 1, g][C - 1:C, :])
            wk_l[c, g] = jnp.concatenate([prev, rk_l[c, g][:C - 1, :]], axis=0)
    for c in range(CPB):
        rs = slice(c * C, (c + 1) * C)
        for g in range(G):
            sl = slice(g * d, (g + 1) * d)
            wkb = wk_l[c, g] * b_all[rs, sl]
            pre = _mmT(jnp.concatenate([wkb, rk_l[c, g]], axis=0), wk_l[c, g])
            Mk_l[c, g] = jnp.where(strict, -pre[:C] * L_all[g], 0.0)
            attn_l[c, g] = pre[C:] * L_all[g]
            X_l[c, g] = jnp.concatenate([v_all[rs, sl], wkb * de_all[g]],
                                        axis=1)
    for c in range(CPB):
        for g in range(G):
            X_l[c, g] = X_l[c, g] + _mm(Mk_l[c, g], X_l[c, g])
    for _ in range(5):
        for c in range(CPB):
            for g in range(G):
                Mk_l[c, g] = _mm(Mk_l[c, g], Mk_l[c, g])
        for c in range(CPB):
            for g in range(G):
                X_l[c, g] = X_l[c, g] + _mm(Mk_l[c, g], X_l[c, g])

    S_vals = [S_all[g * d:(g + 1) * d, :] for g in range(G)]
    for c in range(CPB):
        rs = slice(c * C, (c + 1) * C)
        vn_c = []
        for g in range(G):
            vn_c.append(X_l[c, g][:, :d] - _mm(X_l[c, g][:, d:], S_vals[g]))
        for g in range(G):
            sl = slice(g * d, (g + 1) * d)
            o_ref[rs, sl] = _mm(
                jnp.concatenate([rk_l[c, g] * de_all[g], attn_l[c, g]], axis=1),
                jnp.concatenate([S_vals[g], vn_c[g]], axis=0))
        for g in range(G):
            S_vals[g] = (ct_ref[hg * G + g] * S_vals[g]
                         + _mTm(wk_l[c, g] * dw_all[g], vn_c[g]))

    S_ref[...] = jnp.concatenate(S_vals, axis=0)
    prev_ref[0:G, :] = jnp.concatenate(
        [rk_l[CPB - 1, g][C - 1:C, :] for g in range(G)], axis=0)


def kernel(out, W_write, W_read, W_beta, decay):
    B, T, D = out.shape
    H = decay.shape[0]
    d = D // H
    N = T // C
    BT = B * T
    f32 = jnp.float32

    x2 = out.reshape(BT, D).astype(f32)

    log_gamma = jnp.log(jax.nn.sigmoid(decay))
    pos = jnp.arange(C, dtype=f32)
    cum = (pos + 1.0) * log_gamma[:, None]
    tril = jnp.tril(jnp.ones((C, C), f32))
    L_mask = jnp.exp((cum[:, :, None] - cum[:, None, :]) * tril) * tril
    decay_exp = jnp.broadcast_to(jnp.exp(cum)[:, :, None], (H, C, d)) + 0.0
    chunk_total = jnp.exp(cum[:, -1])
    dw = jnp.broadcast_to(jnp.exp(cum[:, -1:] - cum)[:, :, None], (H, C, d)) + 0.0
    eh = jnp.repeat(jnp.eye(H, dtype=f32), d, axis=1)

    n_tiles = BT // TM
    v_flat, bexp_flat = pl.pallas_call(
        _proj_in_body,
        grid=(n_tiles,),
        in_specs=[
            pl.BlockSpec((TM, D), lambda i: (i, 0)),
            pl.BlockSpec((D, D), lambda i: (0, 0)),
            pl.BlockSpec((H, D), lambda i: (0, 0)),
            pl.BlockSpec((H, D), lambda i: (0, 0)),
        ],
        out_specs=[
            pl.BlockSpec((TM, D), lambda i: (i, 0)),
            pl.BlockSpec((TM, D), lambda i: (i, 0)),
        ],
        out_shape=[
            jax.ShapeDtypeStruct((BT, D), f32),
            jax.ShapeDtypeStruct((BT, D), f32),
        ],
        compiler_params=pltpu.CompilerParams(
            dimension_semantics=("parallel",)),
        name="hebbian_proj_in",
    )(x2, W_write, W_beta, eh)

    BH = B * H
    scan_body = functools.partial(_scan_body, H=H, d=d)
    NB = N // CPB
    row_map = lambda p, n: (p // (H // G) * NB + n, jax.lax.rem(p, H // G))
    hd_map = lambda p, n: (jax.lax.rem(p, H // G), 0, 0)
    o_flat = pl.pallas_call(
        scan_body,
        grid=(BH // G, NB),
        in_specs=[
            pl.BlockSpec((CPB * C, G * d), row_map),
            pl.BlockSpec((CPB * C, G * d), row_map),
            pl.BlockSpec((CPB * C, G * d), row_map),
            pl.BlockSpec((G, C, C), hd_map),
            pl.BlockSpec((G, C, d), hd_map),
            pl.BlockSpec((G, C, d), hd_map),
            pl.BlockSpec(memory_space=pltpu.SMEM),
        ],
        out_specs=pl.BlockSpec((CPB * C, G * d), row_map),
        out_shape=jax.ShapeDtypeStruct((BT, D), f32),
        scratch_shapes=[
            pltpu.VMEM((G * d, d), f32),
            pltpu.VMEM((8, d), f32),
        ],
        compiler_params=pltpu.CompilerParams(
            dimension_semantics=("parallel", "arbitrary")),
        name="hebbian_scan",
    )(x2, v_flat, bexp_flat, L_mask, decay_exp, dw, chunk_total)

    y = pl.pallas_call(
        _proj_out_body,
        grid=(n_tiles,),
        in_specs=[
            pl.BlockSpec((TM, D), lambda i: (i, 0)),
            pl.BlockSpec((TM, D), lambda i: (i, 0)),
            pl.BlockSpec((D, D), lambda i: (0, 0)),
        ],
        out_specs=pl.BlockSpec((TM, D), lambda i: (i, 0)),
        out_shape=jax.ShapeDtypeStruct((BT, D), f32),
        compiler_params=pltpu.CompilerParams(
            dimension_semantics=("parallel",)),
        name="hebbian_proj_out",
    )(o_flat, x2, W_read)

    return y.reshape(B, T, D).astype(out.dtype)
